# baseline (device time: 35012 ns/iter reference)
import jax
import jax.numpy as jnp
from jax import lax
from jax.experimental import pallas as pl
from jax.experimental.pallas import tpu as pltpu

N_DEV = 4
B_LOC = 2
SQ = 128
SKV = 128
HQ = 16
H_LOC = 4
DH = 64
D_MODEL = 512
D_QKV = 256


def kernel(x, Wq, K_ext, V_ext, Wo):
    my = lax.axis_index("i")

    x_b = x.astype(jnp.bfloat16)
    wq_b = Wq.astype(jnp.bfloat16)
    wo_b = Wo.astype(jnp.bfloat16)
    k_loc = lax.dynamic_slice_in_dim(K_ext, my * B_LOC, B_LOC, axis=0)
    v_loc = lax.dynamic_slice_in_dim(V_ext, my * B_LOC, B_LOC, axis=0)
    k_loc = jnp.transpose(k_loc, (0, 2, 1, 3)).astype(jnp.bfloat16)
    v_loc = jnp.transpose(v_loc, (0, 2, 1, 3)).astype(jnp.bfloat16)

    def body(x_ref, wq_ref, k_ref, v_ref, wo_ref, out_ref,
             wq_comm, wo_comm, wq_ssem, wq_rsem, wo_ssem, wo_rsem):
        my_pos = lax.axis_index("i")
        left = lax.rem(my_pos + N_DEV - 1, N_DEV)
        right = lax.rem(my_pos + 1, N_DEV)

        barrier_sem = pltpu.get_barrier_semaphore()
        for nbr in (left, right):
            pl.semaphore_signal(
                barrier_sem, inc=1,
                device_id=(nbr,), device_id_type=pl.DeviceIdType.MESH,
            )
        pl.semaphore_wait(barrier_sem, 2)

        qb = lax.broadcasted_iota(jnp.int32, (SQ, SKV), 0) // 64
        kb = lax.broadcasted_iota(jnp.int32, (SQ, SKV), 1) // 64
        mask = (qb == kb) | (kb == 0) | (lax.rem(qb + kb, 3) == 0)
        neg = jnp.float32(-1e9)

        def compute_group(h):
            origin = lax.rem(my_pos + N_DEV - h, N_DEV)
            h0 = origin * H_LOC
            wq_g = wq_comm[h]
            wo_g = wo_comm[h]
            for b in range(B_LOC):
                xb = x_ref[b]
                q_all = lax.dot_general(
                    xb, wq_g, (((1,), (0,)), ((), ())),
                    preferred_element_type=jnp.float32,
                ).astype(jnp.bfloat16)
                k_slab = k_ref[b, pl.ds(h0, H_LOC)]
                v_slab = v_ref[b, pl.ds(h0, H_LOC)]
                ctx_parts = []
                for hh in range(H_LOC):
                    q = q_all[:, hh * DH:(hh + 1) * DH]
                    scores = lax.dot_general(
                        q, k_slab[hh], (((1,), (1,)), ((), ())),
                        preferred_element_type=jnp.float32,
                    ) * 0.125
                    scores = jnp.where(mask, scores, neg)
                    m = jnp.max(scores, axis=-1, keepdims=True)
                    w = jnp.exp(scores - m)
                    w = w / jnp.sum(w, axis=-1, keepdims=True)
                    ctx_parts.append(lax.dot_general(
                        w.astype(jnp.bfloat16), v_slab[hh],
                        (((1,), (0,)), ((), ())),
                        preferred_element_type=jnp.float32,
                    ).astype(jnp.bfloat16))
                ctx = jnp.concatenate(ctx_parts, axis=1)
                partial = lax.dot_general(
                    ctx, wo_g, (((1,), (0,)), ((), ())),
                    preferred_element_type=jnp.float32,
                )
                if h == 0:
                    out_ref[b] = partial
                else:
                    out_ref[b] = out_ref[b] + partial

        wq_comm[0] = wq_ref[:, :]
        wo_comm[0] = wo_ref[:, :]

        for h in range(N_DEV - 1):
            wq_rdma = pltpu.make_async_remote_copy(
                src_ref=wq_comm.at[h], dst_ref=wq_comm.at[h + 1],
                send_sem=wq_ssem.at[h], recv_sem=wq_rsem.at[h],
                device_id=(right,), device_id_type=pl.DeviceIdType.MESH,
            )
            wo_rdma = pltpu.make_async_remote_copy(
                src_ref=wo_comm.at[h], dst_ref=wo_comm.at[h + 1],
                send_sem=wo_ssem.at[h], recv_sem=wo_rsem.at[h],
                device_id=(right,), device_id_type=pl.DeviceIdType.MESH,
            )
            wq_rdma.start()
            wo_rdma.start()
            compute_group(h)
            wq_rdma.wait()
            wo_rdma.wait()
        compute_group(N_DEV - 1)

    return pl.pallas_call(
        body,
        out_shape=jax.ShapeDtypeStruct((B_LOC, SQ, D_MODEL), jnp.float32),
        in_specs=[pl.BlockSpec(memory_space=pltpu.VMEM)] * 5,
        out_specs=pl.BlockSpec(memory_space=pltpu.VMEM),
        scratch_shapes=[
            pltpu.VMEM((N_DEV, D_MODEL, D_QKV), jnp.bfloat16),
            pltpu.VMEM((N_DEV, D_QKV, D_MODEL), jnp.bfloat16),
            pltpu.SemaphoreType.DMA((N_DEV - 1,)),
            pltpu.SemaphoreType.DMA((N_DEV - 1,)),
            pltpu.SemaphoreType.DMA((N_DEV - 1,)),
            pltpu.SemaphoreType.DMA((N_DEV - 1,)),
        ],
        compiler_params=pltpu.CompilerParams(collective_id=0),
    )(x_b, wq_b, k_loc, v_loc, wo_b)


# device time: 25826 ns/iter; 1.3557x vs baseline; 1.3557x over previous
import jax
import jax.numpy as jnp
from jax import lax
from jax.experimental import pallas as pl
from jax.experimental.pallas import tpu as pltpu

N_DEV = 4
B_LOC = 2
SQ = 128
SKV = 128
H_LOC = 4
H_HALF = 2
DH = 64
D_MODEL = 512
D_HALF = H_HALF * DH


def kernel(x, Wq, K_ext, V_ext, Wo):
    my = lax.axis_index("i")

    x_b = x.astype(jnp.bfloat16).reshape(B_LOC * SQ, D_MODEL)
    wq_b = Wq.astype(jnp.bfloat16)
    wo_b = Wo.astype(jnp.bfloat16)
    k_loc = lax.dynamic_slice_in_dim(K_ext, my * B_LOC, B_LOC, axis=0)
    v_loc = lax.dynamic_slice_in_dim(V_ext, my * B_LOC, B_LOC, axis=0)
    k_loc = jnp.transpose(k_loc, (0, 2, 1, 3)).astype(jnp.bfloat16)
    v_loc = jnp.transpose(v_loc, (0, 2, 1, 3)).astype(jnp.bfloat16)

    def body(x_ref, wq_ref, k_ref, v_ref, wo_ref, out_ref,
             wqR, woR, wqL, woL,
             wqR_ss, wqR_rs, woR_ss, woR_rs,
             wqL_ss, wqL_rs, woL_ss, woL_rs):
        my_pos = lax.axis_index("i")
        left = lax.rem(my_pos + N_DEV - 1, N_DEV)
        right = lax.rem(my_pos + 1, N_DEV)

        barrier_sem = pltpu.get_barrier_semaphore()
        for nbr in (left, right):
            pl.semaphore_signal(
                barrier_sem, inc=1,
                device_id=(nbr,), device_id_type=pl.DeviceIdType.MESH,
            )
        pl.semaphore_wait(barrier_sem, 2)

        qb = lax.broadcasted_iota(jnp.int32, (SQ, SKV), 0) // 64
        kb = lax.broadcasted_iota(jnp.int32, (SQ, SKV), 1) // 64
        mask = (qb == kb) | (kb == 0) | (lax.rem(qb + kb, 3) == 0)
        neg = jnp.float32(-1e9)

        def compute_half(wq_comm, wo_comm, h, origin, head_off, first=False):
            h0 = origin * H_LOC + head_off
            wq_g = wq_comm[h]
            wo_g = wo_comm[h]
            q_all = lax.dot_general(
                x_ref[:, :], wq_g, (((1,), (0,)), ((), ())),
                preferred_element_type=jnp.float32,
            ).astype(jnp.bfloat16)
            ctx_rows = []
            for b in range(B_LOC):
                k_slab = k_ref[b, pl.ds(h0, H_HALF)]
                v_slab = v_ref[b, pl.ds(h0, H_HALF)]
                ctx_parts = []
                for hh in range(H_HALF):
                    q = q_all[b * SQ:(b + 1) * SQ, hh * DH:(hh + 1) * DH]
                    scores = lax.dot_general(
                        q, k_slab[hh], (((1,), (1,)), ((), ())),
                        preferred_element_type=jnp.float32,
                    ) * 0.125
                    w = jnp.exp(jnp.where(mask, scores, neg))
                    r = 1.0 / jnp.sum(w, axis=-1, keepdims=True)
                    ctx = lax.dot_general(
                        w.astype(jnp.bfloat16), v_slab[hh],
                        (((1,), (0,)), ((), ())),
                        preferred_element_type=jnp.float32,
                    ) * r
                    ctx_parts.append(ctx.astype(jnp.bfloat16))
                ctx_rows.append(jnp.concatenate(ctx_parts, axis=1))
            ctx_all = jnp.concatenate(ctx_rows, axis=0)
            partial = lax.dot_general(
                ctx_all, wo_g, (((1,), (0,)), ((), ())),
                preferred_element_type=jnp.float32,
            )
            if first:
                out_ref[:, :] = partial
            else:
                out_ref[:, :] = out_ref[:, :] + partial

        wqR[0] = wq_ref[:, 0:D_HALF]
        woR[0] = wo_ref[0:D_HALF, :]
        wqL[0] = wq_ref[:, D_HALF:2 * D_HALF]
        woL[0] = wo_ref[D_HALF:2 * D_HALF, :]

        for h in range(N_DEV - 1):
            rdmas = []
            for (wq_c, wo_c, ss_q, rs_q, ss_o, rs_o, dst) in (
                (wqR, woR, wqR_ss, wqR_rs, woR_ss, woR_rs, right),
                (wqL, woL, wqL_ss, wqL_rs, woL_ss, woL_rs, left),
            ):
                for (buf, ss, rs) in ((wq_c, ss_q, rs_q), (wo_c, ss_o, rs_o)):
                    rdma = pltpu.make_async_remote_copy(
                        src_ref=buf.at[h], dst_ref=buf.at[h + 1],
                        send_sem=ss.at[h], recv_sem=rs.at[h],
                        device_id=(dst,), device_id_type=pl.DeviceIdType.MESH,
                    )
                    rdma.start()
                    rdmas.append(rdma)
            compute_half(wqR, woR, h, lax.rem(my_pos + N_DEV - h, N_DEV), 0,
                         first=(h == 0))
            compute_half(wqL, woL, h, lax.rem(my_pos + h, N_DEV), H_HALF)
            for rdma in rdmas:
                rdma.wait()
        h = N_DEV - 1
        compute_half(wqR, woR, h, lax.rem(my_pos + N_DEV - h, N_DEV), 0)
        compute_half(wqL, woL, h, lax.rem(my_pos + h, N_DEV), H_HALF)

    out = pl.pallas_call(
        body,
        out_shape=jax.ShapeDtypeStruct((B_LOC * SQ, D_MODEL), jnp.float32),
        in_specs=[pl.BlockSpec(memory_space=pltpu.VMEM)] * 5,
        out_specs=pl.BlockSpec(memory_space=pltpu.VMEM),
        scratch_shapes=[
            pltpu.VMEM((N_DEV, D_MODEL, D_HALF), jnp.bfloat16),
            pltpu.VMEM((N_DEV, D_HALF, D_MODEL), jnp.bfloat16),
            pltpu.VMEM((N_DEV, D_MODEL, D_HALF), jnp.bfloat16),
            pltpu.VMEM((N_DEV, D_HALF, D_MODEL), jnp.bfloat16),
        ] + [pltpu.SemaphoreType.DMA((N_DEV - 1,))] * 8,
        compiler_params=pltpu.CompilerParams(collective_id=0),
    )(x_b, wq_b, k_loc, v_loc, wo_b)
    return out.reshape(B_LOC, SQ, D_MODEL)


# device time: 23099 ns/iter; 1.5157x vs baseline; 1.1181x over previous
import jax
import jax.numpy as jnp
from jax import lax
from jax.experimental import pallas as pl
from jax.experimental.pallas import tpu as pltpu

N_DEV = 4
B_LOC = 2
SQ = 128
SKV = 128
H_LOC = 4
H_HALF = 2
DH = 64
D_MODEL = 512
D_HALF = H_HALF * DH


def kernel(x, Wq, K_ext, V_ext, Wo):
    my = lax.axis_index("i")

    x_b = x.astype(jnp.bfloat16).reshape(B_LOC * SQ, D_MODEL)
    n_b_glob = K_ext.shape[0]
    k2 = K_ext.reshape(n_b_glob, SQ, 16 * DH)
    v2 = V_ext.reshape(n_b_glob, SQ, 16 * DH)
    k_loc = lax.dynamic_slice_in_dim(k2, my * B_LOC, B_LOC, axis=0).astype(jnp.bfloat16)
    v_loc = lax.dynamic_slice_in_dim(v2, my * B_LOC, B_LOC, axis=0).astype(jnp.bfloat16)

    def body(x_ref, wq_ref, k_ref, v_ref, wo_ref, out_ref,
             wqR, woR, wqL, woL,
             wqR_ss, wqR_rs, woR_ss, woR_rs,
             wqL_ss, wqL_rs, woL_ss, woL_rs):
        my_pos = lax.axis_index("i")
        left = lax.rem(my_pos + N_DEV - 1, N_DEV)
        right = lax.rem(my_pos + 1, N_DEV)

        barrier_sem = pltpu.get_barrier_semaphore()
        for nbr in (left, right):
            pl.semaphore_signal(
                barrier_sem, inc=1,
                device_id=(nbr,), device_id_type=pl.DeviceIdType.MESH,
            )
        pl.semaphore_wait(barrier_sem, 2)

        qb = lax.broadcasted_iota(jnp.int32, (SQ, SKV), 0) // 64
        kb = lax.broadcasted_iota(jnp.int32, (SQ, SKV), 1) // 64
        mask = (qb == kb) | (kb == 0) | (lax.rem(qb + kb, 3) == 0)
        neg = jnp.float32(-1e9)

        def compute_half(wq_comm, wo_comm, h, origin, head_off, first=False):
            c0 = (origin * H_LOC + head_off) * DH
            wq_g = wq_comm[h]
            wo_g = wo_comm[h]
            q_all = lax.dot_general(
                x_ref[:, :], wq_g, (((1,), (0,)), ((), ())),
                preferred_element_type=jnp.float32,
            ).astype(jnp.bfloat16)
            ctx_rows = []
            for b in range(B_LOC):
                k_pair = k_ref[b, :, pl.ds(c0, H_HALF * DH)]
                v_pair = v_ref[b, :, pl.ds(c0, H_HALF * DH)]
                ctx_parts = []
                for hh in range(H_HALF):
                    q = q_all[b * SQ:(b + 1) * SQ, hh * DH:(hh + 1) * DH]
                    scores = lax.dot_general(
                        q, k_pair[:, hh * DH:(hh + 1) * DH],
                        (((1,), (1,)), ((), ())),
                        preferred_element_type=jnp.float32,
                    ) * 0.125
                    w = jnp.exp(jnp.where(mask, scores, neg))
                    r = 1.0 / jnp.sum(w, axis=-1, keepdims=True)
                    ctx = lax.dot_general(
                        w.astype(jnp.bfloat16), v_pair[:, hh * DH:(hh + 1) * DH],
                        (((1,), (0,)), ((), ())),
                        preferred_element_type=jnp.float32,
                    ) * r
                    ctx_parts.append(ctx.astype(jnp.bfloat16))
                ctx_rows.append(jnp.concatenate(ctx_parts, axis=1))
            ctx_all = jnp.concatenate(ctx_rows, axis=0)
            partial = lax.dot_general(
                ctx_all, wo_g, (((1,), (0,)), ((), ())),
                preferred_element_type=jnp.float32,
            )
            if first:
                out_ref[:, :] = partial
            else:
                out_ref[:, :] = out_ref[:, :] + partial

        wqR[0] = wq_ref[:, 0:D_HALF].astype(jnp.bfloat16)
        woR[0] = wo_ref[0:D_HALF, :].astype(jnp.bfloat16)
        wqL[0] = wq_ref[:, D_HALF:2 * D_HALF].astype(jnp.bfloat16)
        woL[0] = wo_ref[D_HALF:2 * D_HALF, :].astype(jnp.bfloat16)

        pending_sends = []
        for h in range(N_DEV - 1):
            rdmas = []
            for (wq_c, wo_c, ss_q, rs_q, ss_o, rs_o, dst) in (
                (wqR, woR, wqR_ss, wqR_rs, woR_ss, woR_rs, right),
                (wqL, woL, wqL_ss, wqL_rs, woL_ss, woL_rs, left),
            ):
                for (buf, ss, rs) in ((wq_c, ss_q, rs_q), (wo_c, ss_o, rs_o)):
                    rdma = pltpu.make_async_remote_copy(
                        src_ref=buf.at[h], dst_ref=buf.at[h + 1],
                        send_sem=ss.at[h], recv_sem=rs.at[h],
                        device_id=(dst,), device_id_type=pl.DeviceIdType.MESH,
                    )
                    rdma.start()
                    rdmas.append(rdma)
            pending_sends.extend(rdmas)
            compute_half(wqR, woR, h, lax.rem(my_pos + N_DEV - h, N_DEV), 0,
                         first=(h == 0))
            compute_half(wqL, woL, h, lax.rem(my_pos + h, N_DEV), H_HALF)
            for rdma in rdmas:
                rdma.wait_recv()
        h = N_DEV - 1
        compute_half(wqR, woR, h, lax.rem(my_pos + N_DEV - h, N_DEV), 0)
        compute_half(wqL, woL, h, lax.rem(my_pos + h, N_DEV), H_HALF)
        for rdma in pending_sends:
            rdma.wait_send()

    out = pl.pallas_call(
        body,
        out_shape=jax.ShapeDtypeStruct((B_LOC * SQ, D_MODEL), jnp.float32),
        in_specs=[pl.BlockSpec(memory_space=pltpu.VMEM)] * 5,
        out_specs=pl.BlockSpec(memory_space=pltpu.VMEM),
        scratch_shapes=[
            pltpu.VMEM((N_DEV, D_MODEL, D_HALF), jnp.bfloat16),
            pltpu.VMEM((N_DEV, D_HALF, D_MODEL), jnp.bfloat16),
            pltpu.VMEM((N_DEV, D_MODEL, D_HALF), jnp.bfloat16),
            pltpu.VMEM((N_DEV, D_HALF, D_MODEL), jnp.bfloat16),
        ] + [pltpu.SemaphoreType.DMA((N_DEV - 1,))] * 8,
        compiler_params=pltpu.CompilerParams(collective_id=0),
    )(x_b, Wq, k_loc, v_loc, Wo)
    return out.reshape(B_LOC, SQ, D_MODEL)


# device time: 19699 ns/iter; 1.7773x vs baseline; 1.1726x over previous
import jax
import jax.numpy as jnp
from jax import lax
from jax.experimental import pallas as pl
from jax.experimental.pallas import tpu as pltpu

N_DEV = 4
B_LOC = 2
SQ = 128
SKV = 128
HQ = 16
H_LOC = 4
DH = 64
D_MODEL = 512
D_GRP = H_LOC * DH
D_HALF = D_GRP // 2


def kernel(x, Wq, K_ext, V_ext, Wo):
    my = lax.axis_index("i")

    x_b = x.astype(jnp.bfloat16).reshape(B_LOC * SQ, D_MODEL)
    n_b_glob = K_ext.shape[0]
    k2 = K_ext.reshape(n_b_glob, SQ, HQ * DH)
    v2 = V_ext.reshape(n_b_glob, SQ, HQ * DH)
    k_loc = lax.dynamic_slice_in_dim(k2, my * B_LOC, B_LOC, axis=0).astype(jnp.bfloat16)
    v_loc = lax.dynamic_slice_in_dim(v2, my * B_LOC, B_LOC, axis=0).astype(jnp.bfloat16)

    def body(x_ref, wq_ref, k_ref, v_ref, wo_ref, out_ref,
             wqR, wqL, woR, woL, ssem, rsem):
        my_pos = lax.axis_index("i")
        left = lax.rem(my_pos + N_DEV - 1, N_DEV)
        right = lax.rem(my_pos + 1, N_DEV)

        barrier_sem = pltpu.get_barrier_semaphore()
        for nbr in (left, right):
            pl.semaphore_signal(
                barrier_sem, inc=1,
                device_id=(nbr,), device_id_type=pl.DeviceIdType.MESH,
            )
        pl.semaphore_wait(barrier_sem, 2)

        qb = lax.broadcasted_iota(jnp.int32, (SQ, SKV), 0) // 64
        kb = lax.broadcasted_iota(jnp.int32, (SQ, SKV), 1) // 64
        mask = (qb == kb) | (kb == 0) | (lax.rem(qb + kb, 3) == 0)
        neg = jnp.float32(-1e9)

        descs = [
            (wqR, 0, 1, right), (woR, 0, 1, right),
            (wqL, 0, 3, left),  (woL, 0, 3, left),
            (wqL, 0, 1, right), (woL, 0, 1, right),
            (wqR, 0, 3, left),  (woR, 0, 3, left),
            (wqR, 1, 2, right), (woR, 1, 2, right),
            (wqL, 3, 2, left),  (woL, 3, 2, left),
        ]
        rdmas = [
            pltpu.make_async_remote_copy(
                src_ref=buf.at[s_slot], dst_ref=buf.at[d_slot],
                send_sem=ssem.at[i], recv_sem=rsem.at[i],
                device_id=(dst,), device_id_type=pl.DeviceIdType.MESH,
            )
            for i, (buf, s_slot, d_slot, dst) in enumerate(descs)
        ]

        def compute_group(slot, origin, first=False):
            c0 = origin * D_GRP
            wq_g = jnp.concatenate([wqR[slot], wqL[slot]], axis=1)
            wo_g = jnp.concatenate([woR[slot], woL[slot]], axis=0)
            q_all = lax.dot_general(
                x_ref[:, :], wq_g, (((1,), (0,)), ((), ())),
                preferred_element_type=jnp.float32,
            ).astype(jnp.bfloat16)
            ctx_rows = []
            for b in range(B_LOC):
                k_grp = k_ref[b, :, pl.ds(c0, D_GRP)]
                v_grp = v_ref[b, :, pl.ds(c0, D_GRP)]
                ctx_parts = []
                for hh in range(H_LOC):
                    q = q_all[b * SQ:(b + 1) * SQ, hh * DH:(hh + 1) * DH]
                    scores = lax.dot_general(
                        q, k_grp[:, hh * DH:(hh + 1) * DH],
                        (((1,), (1,)), ((), ())),
                        preferred_element_type=jnp.float32,
                    ) * 0.125
                    w = jnp.exp(jnp.where(mask, scores, neg))
                    r = 1.0 / jnp.sum(w, axis=-1, keepdims=True)
                    ctx = lax.dot_general(
                        w.astype(jnp.bfloat16), v_grp[:, hh * DH:(hh + 1) * DH],
                        (((1,), (0,)), ((), ())),
                        preferred_element_type=jnp.float32,
                    ) * r
                    ctx_parts.append(ctx.astype(jnp.bfloat16))
                ctx_rows.append(jnp.concatenate(ctx_parts, axis=1))
            ctx_all = jnp.concatenate(ctx_rows, axis=0)
            partial = lax.dot_general(
                ctx_all, wo_g, (((1,), (0,)), ((), ())),
                preferred_element_type=jnp.float32,
            )
            if first:
                out_ref[:, :] = partial
            else:
                out_ref[:, :] = out_ref[:, :] + partial

        wqR[0] = wq_ref[:, 0:D_HALF].astype(jnp.bfloat16)
        wqL[0] = wq_ref[:, D_HALF:D_GRP].astype(jnp.bfloat16)
        woR[0] = wo_ref[0:D_HALF, :].astype(jnp.bfloat16)
        woL[0] = wo_ref[D_HALF:D_GRP, :].astype(jnp.bfloat16)

        for i in (0, 2, 1, 3, 4, 6, 5, 7):
            rdmas[i].start()

        compute_group(0, my_pos, first=True)

        rdmas[0].wait_recv()
        rdmas[1].wait_recv()
        rdmas[8].start()
        rdmas[9].start()
        rdmas[2].wait_recv()
        rdmas[3].wait_recv()
        rdmas[10].start()
        rdmas[11].start()

        rdmas[4].wait_recv()
        rdmas[5].wait_recv()
        compute_group(1, left)
        rdmas[6].wait_recv()
        rdmas[7].wait_recv()
        compute_group(3, right)
        for i in (8, 9, 10, 11):
            rdmas[i].wait_recv()
        compute_group(2, lax.rem(my_pos + 2, N_DEV))

        for rdma in rdmas:
            rdma.wait_send()

    out = pl.pallas_call(
        body,
        out_shape=jax.ShapeDtypeStruct((B_LOC * SQ, D_MODEL), jnp.float32),
        in_specs=[pl.BlockSpec(memory_space=pltpu.VMEM)] * 5,
        out_specs=pl.BlockSpec(memory_space=pltpu.VMEM),
        scratch_shapes=[
            pltpu.VMEM((N_DEV, D_MODEL, D_HALF), jnp.bfloat16),
            pltpu.VMEM((N_DEV, D_MODEL, D_HALF), jnp.bfloat16),
            pltpu.VMEM((N_DEV, D_HALF, D_MODEL), jnp.bfloat16),
            pltpu.VMEM((N_DEV, D_HALF, D_MODEL), jnp.bfloat16),
            pltpu.SemaphoreType.DMA((12,)),
            pltpu.SemaphoreType.DMA((12,)),
        ],
        compiler_params=pltpu.CompilerParams(collective_id=0),
    )(x_b, Wq, k_loc, v_loc, Wo)
    return out.reshape(B_LOC, SQ, D_MODEL)
